# Initial kernel scaffold; baseline (speedup 1.0000x reference)
import jax
import jax.numpy as jnp
from jax.experimental import pallas as pl


def kernel(x, enc_w1, enc_b1, enc_w2, enc_b2, dec_w1, dec_b1, dec_w2, dec_b2, codebook):
    raise NotImplementedError("write your pallas kernel here")


# speedup vs baseline: 4.6638x; 4.6638x over previous
import jax
import jax.numpy as jnp
import numpy as np
from jax.experimental import pallas as pl
from jax.experimental.pallas import tpu as pltpu

_F32 = jnp.float32
_BF16 = jnp.bfloat16
_SQRT_2 = np.float32(np.sqrt(2.0))
_NEG_INF = np.float32(-np.inf)


def _gelu(x):
    return x * (jax.lax.erf(x / _SQRT_2) + jnp.float32(1.0)) * jnp.float32(0.5)


def _tree_sum(terms):
    while len(terms) > 1:
        nxt = []
        for i in range(0, len(terms) - 1, 2):
            nxt.append(terms[i] + terms[i + 1])
        if len(terms) % 2:
            nxt.append(terms[-1])
        terms = nxt
    return terms[0]


def _pad1_2d(x):
    h, w = x.shape
    zc = jnp.zeros((h, 1), _F32)
    x = jnp.concatenate([zc, x, zc], axis=1)
    zr = jnp.zeros((1, w + 2), _F32)
    return jnp.concatenate([zr, x, zr], axis=0)


def _pool2_rows(x):
    h, w = x.shape
    hp = (h + 2) // 2 * 2
    rows = [jnp.full((1, w), _NEG_INF), x]
    if hp - 1 - h > 0:
        rows.append(jnp.full((hp - 1 - h, w), _NEG_INF))
    xr = jnp.concatenate(rows, axis=0).reshape(hp // 2, 2, w)
    return jnp.max(xr, axis=1)


def _pool2_2d(x):
    return _pool2_rows(_pool2_rows(x).T).T


def _up2_rows(x):
    m, w = x.shape
    return jnp.broadcast_to(x[:, None, :], (m, 2, w)).reshape(2 * m, w)


def _up2_2d(x):
    return _up2_rows(_up2_rows(x).T).T


def _conv_terms(xb, order):
    c_, h, w = xb.shape
    xs = {}
    for c in range(c_):
        xp = _pad1_2d(xb[c])
        for kh in range(3):
            for kw in range(3):
                xs[(c, kh, kw)] = xp[kh:kh + h, kw:kw + w]
    return xs


def _enc1_kernel(x_ref, w_ref, b_ref, o_ref):
    xb = x_ref[0].astype(_F32)
    order = [(c, kh, kw) for kh in range(3) for kw in range(3) for c in range(3)]
    xs = _conv_terms(xb, order)
    views = [xs[key] for key in order]

    def body(o, _):
        terms = [w_ref[o, j] * views[j] for j in range(len(views))]
        y = _tree_sum(terms) + b_ref[o]
        g = _gelu(_pool2_2d(y))
        o_ref[0, o] = g.astype(_BF16)
        return 0

    jax.lax.fori_loop(0, 16, body, 0)


def _enc2_kernel(h_ref, w_ref, b_ref, o_ref):
    xb = h_ref[0].astype(_F32)
    order = [(c, kh, kw) for c in range(16) for kh in range(3) for kw in range(3)]
    xs = _conv_terms(xb, order)

    def prod(c, t):
        return w_ref[0, c * 9 + t] * xs[(c, t // 3, t % 3)]

    parts = []
    for i in range(4):
        pc = [prod(c, 2 * i) + prod(c, 2 * i + 1) for c in range(16)]
        parts.append(_tree_sum(pc))
    parts.append(_tree_sum([prod(c, 8) for c in range(16)]))
    y = parts[0]
    for p in parts[1:]:
        y = y + p
    y = y + b_ref[0]
    o_ref[0, 0] = _pool2_2d(y)


def _vq_kernel(z_ref, cb_ref, idx_ref, q_ref, loss_ref):
    z = z_ref[...]
    zsq = z * z

    def body(k, carry):
        dmin, kmin, qv = carry
        e = cb_ref[k]
        esq = e * e
        d = (zsq + esq) - z * (jnp.float32(2.0) * e)
        better = d < dmin
        return (jnp.where(better, d, dmin),
                jnp.where(better, k, kmin),
                jnp.where(better, e, qv))

    init = (jnp.full(z.shape, jnp.float32(jnp.inf)),
            jnp.zeros(z.shape, jnp.int32),
            jnp.zeros(z.shape, _F32))
    dmin, kmin, qv = jax.lax.fori_loop(0, 1024, body, init)
    diff = qv - z
    idx_ref[...] = kmin
    q_ref[...] = z + diff
    loss_ref[0, 0] = jnp.sum(diff * diff)


def _dec1_kernel(q_ref, w_ref, b_ref, o_ref):
    qf = q_ref[0, 0].astype(_F32)
    u = _up2_2d(qf)[None]
    order = [(0, kh, kw) for kh in range(3) for kw in range(3)]
    xs = _conv_terms(u, order)
    views = [xs[key] for key in order]

    def body(o, _):
        terms = [w_ref[o, j] * views[j] for j in range(len(views))]
        y = _tree_sum(terms) + b_ref[o]
        o_ref[0, o] = _gelu(y).astype(_BF16)
        return 0

    jax.lax.fori_loop(0, 16, body, 0)


def _dec2_kernel(h_ref, w_ref, b_ref, o_ref):
    def body(c, acc):
        xp = _pad1_2d(_up2_2d(h_ref[0, c].astype(_F32)))
        terms = [w_ref[0, c * 9 + kh * 3 + kw] * xp[kh:kh + 388, kw:kw + 388]
                 for kh in range(3) for kw in range(3)]
        return acc + _tree_sum(terms)

    acc = jax.lax.fori_loop(0, 16, body, jnp.zeros((388, 388), _F32))
    y = acc + b_ref[0]
    o_ref[0, 0] = jnp.clip(y, -1.0, 1.0)


def _smem_spec():
    return pl.BlockSpec(memory_space=pltpu.SMEM)


def kernel(x, enc_w1, enc_b1, enc_w2, enc_b2, dec_w1, dec_b1, dec_w2, dec_b2,
           codebook):
    b = x.shape[0]
    xb = x.astype(_BF16)
    def _rp(w):
        return jax.lax.reduce_precision(w, exponent_bits=8, mantissa_bits=7)
    w1p = _rp(enc_w1).transpose(0, 2, 3, 1).reshape(16, 27)
    w2p = _rp(enc_w2).reshape(1, 144)
    w3p = _rp(dec_w1).reshape(16, 9)
    w4p = _rp(dec_w2).reshape(1, 144)
    cb = codebook.reshape(1024)

    h1 = pl.pallas_call(
        _enc1_kernel,
        grid=(b,),
        in_specs=[pl.BlockSpec((1, 3, 384, 384), lambda i: (i, 0, 0, 0)),
                  _smem_spec(), _smem_spec()],
        out_specs=pl.BlockSpec((1, 16, 193, 193), lambda i: (i, 0, 0, 0)),
        out_shape=jax.ShapeDtypeStruct((b, 16, 193, 193), _BF16),
    )(xb, w1p, enc_b1)

    z = pl.pallas_call(
        _enc2_kernel,
        grid=(b,),
        in_specs=[pl.BlockSpec((1, 16, 193, 193), lambda i: (i, 0, 0, 0)),
                  _smem_spec(), _smem_spec()],
        out_specs=pl.BlockSpec((1, 1, 97, 97), lambda i: (i, 0, 0, 0)),
        out_shape=jax.ShapeDtypeStruct((b, 1, 97, 97), _F32),
    )(h1, w2p, enc_b2)

    n = 97 * 97
    npad = 74 * 128 - n
    z2 = z.reshape(b, n)
    zp = jnp.concatenate([z2, jnp.full((b, npad), codebook[0, 0], _F32)], axis=1)
    zp = zp.reshape(b * 74, 128)

    idx, q2, loss_sum = pl.pallas_call(
        _vq_kernel,
        in_specs=[pl.BlockSpec(memory_space=pltpu.VMEM), _smem_spec()],
        out_specs=[pl.BlockSpec(memory_space=pltpu.VMEM),
                   pl.BlockSpec(memory_space=pltpu.VMEM),
                   _smem_spec()],
        out_shape=[jax.ShapeDtypeStruct((b * 74, 128), jnp.int32),
                   jax.ShapeDtypeStruct((b * 74, 128), _F32),
                   jax.ShapeDtypeStruct((1, 1), _F32)],
    )(zp, cb)

    indices_map = idx.reshape(b, 74 * 128)[:, :n].reshape(b, 97, 97)
    qmap = q2.reshape(b, 74 * 128)[:, :n].reshape(b, 1, 97, 97).astype(_BF16)
    commit_loss = loss_sum[0, 0] / jnp.float32(b * n)

    d1 = pl.pallas_call(
        _dec1_kernel,
        grid=(b,),
        in_specs=[pl.BlockSpec((1, 1, 97, 97), lambda i: (i, 0, 0, 0)),
                  _smem_spec(), _smem_spec()],
        out_specs=pl.BlockSpec((1, 16, 194, 194), lambda i: (i, 0, 0, 0)),
        out_shape=jax.ShapeDtypeStruct((b, 16, 194, 194), _BF16),
    )(qmap, w3p, dec_b1)

    recon = pl.pallas_call(
        _dec2_kernel,
        grid=(b,),
        in_specs=[pl.BlockSpec((1, 16, 194, 194), lambda i: (i, 0, 0, 0)),
                  _smem_spec(), _smem_spec()],
        out_specs=pl.BlockSpec((1, 1, 388, 388), lambda i: (i, 0, 0, 0)),
        out_shape=jax.ShapeDtypeStruct((b, 1, 388, 388), _F32),
    )(d1, w4p, dec_b2)

    return recon, indices_map, commit_loss
